# baseline (speedup 1.0000x reference)
import functools

import jax
import jax.numpy as jnp
from jax import lax
from jax.experimental import pallas as pl
from jax.experimental.pallas import tpu as pltpu
from jax.experimental.pallas import tpu_sc as plsc

_IDBUF = 4
_GBUF = 4


def _emb_lookup(idsT, tabT, nc, nw):
    H, B = idsT.shape
    D, V = tabT.shape
    d_per_w = D // nw
    mesh = plsc.VectorSubcoreMesh(core_axis_name="c", subcore_axis_name="s")

    @functools.partial(
        pl.kernel,
        mesh=mesh,
        out_type=jax.ShapeDtypeStruct((H, D, B), jnp.float32),
        compiler_params=pltpu.CompilerParams(use_tc_tiling_on_sc=False),
        scratch_types=[
            pltpu.VMEM((_IDBUF, B), jnp.int32),
            pltpu.VMEM((_GBUF, B), jnp.float32),
            pltpu.SemaphoreType.DMA,
            pltpu.SemaphoreType.DMA,
            pltpu.SemaphoreType.DMA,
        ],
    )
    def emb(ids_hbm, tab_hbm, out_hbm, idv, gbuf, idsem, gsem, osem):
        wid = lax.axis_index("s") * nc + lax.axis_index("c")
        d0 = wid * d_per_w

        def id_copy(h):
            return pltpu.make_async_copy(
                ids_hbm.at[h], idv.at[lax.rem(h, _IDBUF)], idsem
            )

        def g_copy(h, dd):
            return pltpu.make_async_copy(
                tab_hbm.at[d0 + dd].at[idv.at[lax.rem(h, _IDBUF)]],
                gbuf.at[lax.rem(h * d_per_w + dd, _GBUF)],
                gsem,
            )

        def o_copy(h, dd):
            return pltpu.make_async_copy(
                gbuf.at[lax.rem(h * d_per_w + dd, _GBUF)],
                out_hbm.at[h, d0 + dd],
                osem,
            )

        for h in range(2):
            id_copy(h).start()

        def body(h):
            id_copy(h).wait()

            @pl.when(h + 2 < H)
            def _():
                id_copy(h + 2).start()

            @pl.when(h >= 2)
            def _():
                for dd in range(d_per_w):
                    o_copy(h - 2, dd).wait()

            for dd in range(d_per_w):
                g_copy(h, dd).start()

            @pl.when(h >= 1)
            def _():
                for dd in range(d_per_w):
                    g_copy(h - 1, dd).wait()
                    o_copy(h - 1, dd).start()

        pl.loop(0, H)(body)

        for dd in range(d_per_w):
            g_copy(H - 1, dd).wait()
            o_copy(H - 1, dd).start()
        for h in (H - 2, H - 1):
            for dd in range(d_per_w):
                o_copy(h, dd).wait()

    return emb(idsT, tabT)


def kernel(ids, table):
    B, H = ids.shape
    V, D = table.shape
    info = plsc.get_sparse_core_info()
    nc, ns = info.num_cores, info.num_subcores
    nw = nc * ns
    idsT = jnp.swapaxes(ids, 0, 1).astype(jnp.int32)
    tabT = jnp.swapaxes(table, 0, 1)
    out = _emb_lookup(idsT, tabT, nc, nw)
    return jnp.transpose(out, (2, 0, 1))


# speedup vs baseline: 7.2470x; 7.2470x over previous
import functools

import jax
import jax.numpy as jnp
from jax import lax
from jax.experimental import pallas as pl
from jax.experimental.pallas import tpu as pltpu
from jax.experimental.pallas import tpu_sc as plsc

_CHUNK = 128
_NBUF = 5


def _emb_lookup(ids_flat, table_pad, n_steps, nc, nw):
    Dp = table_pad.shape[1]
    N = ids_flat.shape[0]
    n_per_w = n_steps * _CHUNK
    mesh = plsc.VectorSubcoreMesh(core_axis_name="c", subcore_axis_name="s")

    @functools.partial(
        pl.kernel,
        mesh=mesh,
        out_type=jax.ShapeDtypeStruct((N, Dp), jnp.float32),
        compiler_params=pltpu.CompilerParams(use_tc_tiling_on_sc=False),
        scratch_types=[
            pltpu.VMEM((n_per_w,), jnp.int32),
            pltpu.VMEM((_NBUF, _CHUNK, Dp), jnp.float32),
            pltpu.SemaphoreType.DMA,
            pltpu.SemaphoreType.DMA,
        ],
    )
    def emb(ids_hbm, table_hbm, out_hbm, idx_v, rows_v, gsem, osem):
        wid = lax.axis_index("s") * nc + lax.axis_index("c")
        base = wid * n_per_w
        pltpu.sync_copy(ids_hbm.at[pl.ds(base, n_per_w)], idx_v)

        def gather_copy(ci, buf):
            return pltpu.make_async_copy(
                table_hbm.at[idx_v.at[pl.ds(ci * _CHUNK, _CHUNK)]],
                rows_v.at[buf],
                gsem,
            )

        def out_copy(ci, buf):
            return pltpu.make_async_copy(
                rows_v.at[buf],
                out_hbm.at[pl.ds(base + ci * _CHUNK, _CHUNK)],
                osem,
            )

        for b in range(_NBUF):
            gather_copy(b, b).start()

        def body(g):
            for i in range(_NBUF):
                j = g + i
                gather_copy(j, i).wait()
                out_copy(j, i).start()
                out_copy(j, i).wait()
                gather_copy(j + _NBUF, i).start()

        pl.loop(0, n_steps - _NBUF, step=_NBUF)(body)

        for i in range(_NBUF):
            j = n_steps - _NBUF + i
            gather_copy(j, i).wait()
            out_copy(j, i).start()
            out_copy(j, i).wait()

    return emb(ids_flat, table_pad)


def kernel(ids, table):
    B, H = ids.shape
    V, D = table.shape
    N = B * H
    info = plsc.get_sparse_core_info()
    nc, ns = info.num_cores, info.num_subcores
    nw = nc * ns
    n_steps = N // (nw * _CHUNK)
    ids_flat = ids.reshape(N).astype(jnp.int32)
    table_pad = jnp.pad(table, ((0, 0), (0, 128 - D)))
    out = _emb_lookup(ids_flat, table_pad, n_steps, nc, nw)
    return out[:, :D].reshape(B, H, D)
